# baseline (speedup 1.0000x reference)
import functools

import jax
import jax.numpy as jnp
from jax import lax
from jax.experimental import pallas as pl
from jax.experimental.pallas import tpu as pltpu
from jax.experimental.pallas import tpu_sc as plsc

N = 4194304
B = 4096

NC = 2
NS = 16
NW = NC * NS
CHUNK = N // NW
PIECE = 16384
NPIECE = CHUNK // PIECE
PROWS = PIECE // 128


def _sc_seg_body(sq_hbm, ids_hbm, out_hbm, vals_v, ids_v, zero_v, accum_sh):
    cid = lax.axis_index("c")
    sid = lax.axis_index("s")
    wid = sid * NC + cid

    def _z(i, _):
        zero_v[pl.ds(i * 16, 16)] = jnp.zeros((16,), jnp.float32)
        return 0

    lax.fori_loop(0, B // 16, _z, 0)

    @pl.when(sid == 0)
    def _():
        pltpu.sync_copy(zero_v, accum_sh)

    plsc.subcore_barrier()

    base = wid * CHUNK

    def _piece(p, _):
        off = pl.multiple_of(base + p * PIECE, PIECE)
        pltpu.sync_copy(sq_hbm.at[pl.ds(off, PIECE)], vals_v)
        pltpu.sync_copy(ids_hbm.at[pl.ds(off, PIECE)], ids_v)
        pltpu.sync_copy(vals_v, accum_sh.at[ids_v], add=True)
        return 0

    lax.fori_loop(0, NPIECE, _piece, 0)

    plsc.subcore_barrier()

    @pl.when(sid == 0)
    def _():
        pltpu.sync_copy(accum_sh, out_hbm.at[cid])


def _sc_call(sq2d, ids2d):
    mesh = plsc.VectorSubcoreMesh(core_axis_name="c", subcore_axis_name="s")
    f = functools.partial(
        pl.kernel,
        out_type=jax.ShapeDtypeStruct((NC, B), jnp.float32),
        mesh=mesh,
        scratch_types=[
            pltpu.VMEM((PIECE,), jnp.float32),
            pltpu.VMEM((PIECE,), jnp.int32),
            pltpu.VMEM((B,), jnp.float32),
            pltpu.VMEM_SHARED((B,), jnp.float32),
        ],
    )(_sc_seg_body)
    return f(sq2d, ids2d)


def kernel(positions, segment_ids, minimum):
    d = positions - minimum
    neg_grad = -2.0 * d
    sq = jnp.sum(d * d, axis=1)
    partial = _sc_call(sq, segment_ids.astype(jnp.int32))
    energies = partial[0] + partial[1]
    stress = jnp.zeros((B, 6), jnp.float32)
    return (energies, neg_grad, stress)


# speedup vs baseline: 2.7143x; 2.7143x over previous
import functools

import jax
import jax.numpy as jnp
from jax import lax
from jax.experimental import pallas as pl
from jax.experimental.pallas import tpu as pltpu
from jax.experimental.pallas import tpu_sc as plsc

N = 4194304
B = 4096

NC = 2
NS = 16
NW = NC * NS
CHUNK = N // NW
PIECE = 16384
NPIECE = CHUNK // PIECE
SLICE = B // NS


def _lane_shift_right(f, lane):
    idx = jnp.maximum(lane - 1, 0)
    dnums = lax.GatherDimensionNumbers(
        offset_dims=(), collapsed_slice_dims=(0,), start_index_map=(0,))
    g = lax.gather(f, idx[:, None], dnums, (1,),
                   mode=lax.GatherScatterMode.PROMISE_IN_BOUNDS)
    return jnp.where(lane == 0, 0.0, g)


def _sc_seg_body(sq_hbm, ids_hbm, out_hbm, vals_v, ids_v, acc_t, merge_v,
                 stage_sh):
    cid = lax.axis_index("c")
    sid = lax.axis_index("s")
    wid = sid * NC + cid

    zeros16 = jnp.zeros((16,), jnp.float32)

    @plsc.parallel_loop(0, B // 16, 1, unroll=8)
    def _z(i):
        acc_t[pl.ds(i * 16, 16)] = zeros16

    base = wid * CHUNK
    lane = lax.iota(jnp.int32, 16)
    is15 = lane == 15

    def _piece(p, _):
        off = pl.multiple_of(base + p * PIECE, PIECE)
        pltpu.sync_copy(sq_hbm.at[pl.ds(off, PIECE)],
                        vals_v.at[pl.ds(0, PIECE)])
        pltpu.sync_copy(ids_hbm.at[pl.ds(off, PIECE)],
                        ids_v.at[pl.ds(0, PIECE)])

        @plsc.parallel_loop(0, PIECE // 16, 1, unroll=8)
        def _vreg(i):
            v = vals_v[pl.ds(i * 16, 16)]
            sids = ids_v[pl.ds(i * 16, 16)]
            nxt = ids_v[pl.ds(i * 16 + 1, 16)]
            c = plsc.cumsum(v)
            m = jnp.logical_or(sids != nxt, is15)
            f = plsc.cummax(jnp.where(m, c, 0.0))
            prev = _lane_shift_right(f, lane)
            plsc.addupdate_scatter(acc_t, [sids], c - prev, mask=m)

        return 0

    lax.fori_loop(0, NPIECE, _piece, 0)

    pltpu.sync_copy(acc_t, stage_sh.at[sid])
    plsc.subcore_barrier()
    pltpu.sync_copy(stage_sh.at[:, pl.ds(sid * SLICE, SLICE)], merge_v)

    def _m(i, _):
        s = merge_v[0, pl.ds(i * 16, 16)]
        for r in range(1, NS):
            s = s + merge_v[r, pl.ds(i * 16, 16)]
        merge_v[0, pl.ds(i * 16, 16)] = s
        return 0

    lax.fori_loop(0, SLICE // 16, _m, 0)
    pltpu.sync_copy(merge_v.at[0], out_hbm.at[cid, pl.ds(sid * SLICE, SLICE)])


def _sc_call(sq, ids):
    mesh = plsc.VectorSubcoreMesh(core_axis_name="c", subcore_axis_name="s")
    f = functools.partial(
        pl.kernel,
        out_type=jax.ShapeDtypeStruct((NC, B), jnp.float32),
        mesh=mesh,
        compiler_params=pltpu.CompilerParams(needs_layout_passes=False),
        scratch_types=[
            pltpu.VMEM((PIECE,), jnp.float32),
            pltpu.VMEM((PIECE + 16,), jnp.int32),
            pltpu.VMEM((B,), jnp.float32),
            pltpu.VMEM((NS, SLICE), jnp.float32),
            pltpu.VMEM_SHARED((NS, B), jnp.float32),
        ],
    )(_sc_seg_body)
    return f(sq, ids)


def kernel(positions, segment_ids, minimum):
    d = positions - minimum
    neg_grad = -2.0 * d
    sq = jnp.sum(d * d, axis=1)
    partial = _sc_call(sq, segment_ids.astype(jnp.int32))
    energies = partial[0] + partial[1]
    stress = jnp.zeros((B, 6), jnp.float32)
    return (energies, neg_grad, stress)


# speedup vs baseline: 2.8609x; 1.0540x over previous
import functools

import jax
import jax.numpy as jnp
from jax import lax
from jax.experimental import pallas as pl
from jax.experimental.pallas import tpu as pltpu
from jax.experimental.pallas import tpu_sc as plsc

N = 4194304
B = 4096

NC = 2
NS = 16
NW = NC * NS
CHUNK = N // NW
PIECE = 16384
NPIECE = CHUNK // PIECE
SLICE = B // NS


def _lane_shift_right(f, lane):
    idx = jnp.maximum(lane - 1, 0)
    dnums = lax.GatherDimensionNumbers(
        offset_dims=(), collapsed_slice_dims=(0,), start_index_map=(0,))
    g = lax.gather(f, idx[:, None], dnums, (1,),
                   mode=lax.GatherScatterMode.PROMISE_IN_BOUNDS)
    return jnp.where(lane == 0, 0.0, g)


def _sc_seg_body(sq_hbm, ids_hbm, out_hbm, vals_v, ids_v, acc_t, merge_v,
                 stage_sh, sems):
    cid = lax.axis_index("c")
    sid = lax.axis_index("s")
    wid = sid * NC + cid

    zeros16 = jnp.zeros((16,), jnp.float32)

    @plsc.parallel_loop(0, B // 16, 1, unroll=8)
    def _z(i):
        acc_t[pl.ds(i * 16, 16)] = zeros16

    base = wid * CHUNK
    lane = lax.iota(jnp.int32, 16)
    is15 = lane == 15

    def _start(k, b):
        off = pl.multiple_of(base + k * PIECE, PIECE)
        pltpu.async_copy(sq_hbm.at[pl.ds(off, PIECE)],
                         vals_v.at[b, pl.ds(0, PIECE)], sems.at[b, 0])
        pltpu.async_copy(ids_hbm.at[pl.ds(off, PIECE)],
                         ids_v.at[b, pl.ds(0, PIECE)], sems.at[b, 1])

    def _wait(k, b):
        off = pl.multiple_of(base + k * PIECE, PIECE)
        pltpu.make_async_copy(sq_hbm.at[pl.ds(off, PIECE)],
                              vals_v.at[b, pl.ds(0, PIECE)], sems.at[b, 0]).wait()
        pltpu.make_async_copy(ids_hbm.at[pl.ds(off, PIECE)],
                              ids_v.at[b, pl.ds(0, PIECE)], sems.at[b, 1]).wait()

    def _compute(b):
        @plsc.parallel_loop(0, PIECE // 16, 1, unroll=8)
        def _vreg(i):
            v = vals_v[b, pl.ds(i * 16, 16)]
            sids = ids_v[b, pl.ds(i * 16, 16)]
            nxt = ids_v[b, pl.ds(i * 16 + 1, 16)]
            c = plsc.cumsum(v)
            m = jnp.logical_or(sids != nxt, is15)
            f = plsc.cummax(jnp.where(m, c, 0.0))
            prev = _lane_shift_right(f, lane)
            plsc.addupdate_scatter(acc_t, [sids], c - prev, mask=m)

    _start(0, 0)

    def _piece(p, _):
        for b in range(2):
            k = p * 2 + b

            @pl.when(k + 1 < NPIECE)
            def _():
                _start(k + 1, 1 - b)

            _wait(k, b)
            _compute(b)
        return 0

    lax.fori_loop(0, NPIECE // 2, _piece, 0)

    pltpu.sync_copy(acc_t, stage_sh.at[sid])
    plsc.subcore_barrier()
    pltpu.sync_copy(stage_sh.at[:, pl.ds(sid * SLICE, SLICE)], merge_v)

    def _m(i, _):
        s = merge_v[0, pl.ds(i * 16, 16)]
        for r in range(1, NS):
            s = s + merge_v[r, pl.ds(i * 16, 16)]
        merge_v[0, pl.ds(i * 16, 16)] = s
        return 0

    lax.fori_loop(0, SLICE // 16, _m, 0)
    pltpu.sync_copy(merge_v.at[0], out_hbm.at[cid, pl.ds(sid * SLICE, SLICE)])


def _sc_call(sq, ids):
    mesh = plsc.VectorSubcoreMesh(core_axis_name="c", subcore_axis_name="s")
    f = functools.partial(
        pl.kernel,
        out_type=jax.ShapeDtypeStruct((NC, B), jnp.float32),
        mesh=mesh,
        compiler_params=pltpu.CompilerParams(needs_layout_passes=False),
        scratch_types=[
            pltpu.VMEM((2, PIECE), jnp.float32),
            pltpu.VMEM((2, PIECE + 16), jnp.int32),
            pltpu.VMEM((B,), jnp.float32),
            pltpu.VMEM((NS, SLICE), jnp.float32),
            pltpu.VMEM_SHARED((NS, B), jnp.float32),
            pltpu.SemaphoreType.DMA((2, 2)),
        ],
    )(_sc_seg_body)
    return f(sq, ids)


def kernel(positions, segment_ids, minimum):
    d = positions - minimum
    neg_grad = -2.0 * d
    sq = jnp.sum(d * d, axis=1)
    partial = _sc_call(sq, segment_ids.astype(jnp.int32))
    energies = partial[0] + partial[1]
    stress = jnp.zeros((B, 6), jnp.float32)
    return (energies, neg_grad, stress)


# speedup vs baseline: 3.1001x; 1.0836x over previous
import functools

import jax
import jax.numpy as jnp
from jax import lax
from jax.experimental import pallas as pl
from jax.experimental.pallas import tpu as pltpu
from jax.experimental.pallas import tpu_sc as plsc

N = 4194304
B = 4096

NC = 2
NS = 16
NW = NC * NS
CHUNK = N // NW
PIECE = 16384
NPIECE = CHUNK // PIECE
SLICE = B // NS


def _lane_shift_right(f, lane):
    idx = jnp.maximum(lane - 1, 0)
    dnums = lax.GatherDimensionNumbers(
        offset_dims=(), collapsed_slice_dims=(0,), start_index_map=(0,))
    g = lax.gather(f, idx[:, None], dnums, (1,),
                   mode=lax.GatherScatterMode.PROMISE_IN_BOUNDS)
    return jnp.where(lane == 0, 0.0, g)


def _sc_seg_body(sq_hbm, ids_hbm, out_hbm, vals_v, ids_v, acc_t, merge_v,
                 stage_sh, sems):
    cid = lax.axis_index("c")
    sid = lax.axis_index("s")
    wid = sid * NC + cid

    zeros16 = jnp.zeros((16,), jnp.float32)

    @plsc.parallel_loop(0, B // 16, 1, unroll=8)
    def _z(i):
        acc_t[pl.ds(i * 16, 16)] = zeros16

    base = wid * CHUNK
    lane = lax.iota(jnp.int32, 16)
    is15 = lane == 15

    def _start(k, b):
        off = pl.multiple_of(base + k * PIECE, PIECE)
        pltpu.async_copy(sq_hbm.at[pl.ds(off, PIECE)],
                         vals_v.at[b, pl.ds(0, PIECE)], sems.at[b, 0])
        pltpu.async_copy(ids_hbm.at[pl.ds(off, PIECE)],
                         ids_v.at[b, pl.ds(0, PIECE)], sems.at[b, 1])

    def _wait(k, b):
        off = pl.multiple_of(base + k * PIECE, PIECE)
        pltpu.make_async_copy(sq_hbm.at[pl.ds(off, PIECE)],
                              vals_v.at[b, pl.ds(0, PIECE)], sems.at[b, 0]).wait()
        pltpu.make_async_copy(ids_hbm.at[pl.ds(off, PIECE)],
                              ids_v.at[b, pl.ds(0, PIECE)], sems.at[b, 1]).wait()

    def _compute(b):
        @plsc.parallel_loop(0, PIECE // 16, 1, unroll=8)
        def _vreg(i):
            v = vals_v[b, pl.ds(i * 16, 16)]
            sids = ids_v[b, pl.ds(i * 16, 16)]
            nxt = ids_v[b, pl.ds(i * 16 + 1, 16)]
            c = plsc.cumsum(v)
            m = jnp.logical_or(sids != nxt, is15)
            f = plsc.cummax(jnp.where(m, c, 0.0))
            prev = _lane_shift_right(f, lane)
            plsc.addupdate_scatter(acc_t, [sids], c - prev, mask=m)

    _start(0, 0)

    def _piece(p, _):
        for b in range(2):
            k = p * 2 + b

            @pl.when(k + 1 < NPIECE)
            def _():
                _start(k + 1, 1 - b)

            _wait(k, b)
            _compute(b)
        return 0

    lax.fori_loop(0, NPIECE // 2, _piece, 0)

    pltpu.sync_copy(acc_t, stage_sh.at[sid])
    plsc.subcore_barrier()
    pltpu.sync_copy(stage_sh.at[:, pl.ds(sid * SLICE, SLICE)], merge_v)

    def _m(i, _):
        s = merge_v[0, pl.ds(i * 16, 16)]
        for r in range(1, NS):
            s = s + merge_v[r, pl.ds(i * 16, 16)]
        merge_v[0, pl.ds(i * 16, 16)] = s
        return 0

    lax.fori_loop(0, SLICE // 16, _m, 0)
    pltpu.sync_copy(merge_v.at[0], out_hbm.at[cid, pl.ds(sid * SLICE, SLICE)])


def _sc_call(sq, ids):
    mesh = plsc.VectorSubcoreMesh(core_axis_name="c", subcore_axis_name="s")
    f = functools.partial(
        pl.kernel,
        out_type=jax.ShapeDtypeStruct((NC, B), jnp.float32),
        mesh=mesh,
        compiler_params=pltpu.CompilerParams(needs_layout_passes=False),
        scratch_types=[
            pltpu.VMEM((2, PIECE), jnp.float32),
            pltpu.VMEM((2, PIECE + 16), jnp.int32),
            pltpu.VMEM((B,), jnp.float32),
            pltpu.VMEM((NS, SLICE), jnp.float32),
            pltpu.VMEM_SHARED((NS, B), jnp.float32),
            pltpu.SemaphoreType.DMA((2, 2)),
        ],
    )(_sc_seg_body)
    return f(sq, ids)


def kernel(positions, segment_ids, minimum):
    d = positions - minimum
    neg_grad = -2.0 * d
    d2 = d * d
    sq = d2[:, 0] + d2[:, 1] + d2[:, 2]
    partial = _sc_call(sq, segment_ids.astype(jnp.int32))
    energies = partial[0] + partial[1]
    stress = jnp.zeros((B, 6), jnp.float32)
    return (energies, neg_grad, stress)
